# baseline (speedup 1.0000x reference)
import functools

import jax
import jax.numpy as jnp
from jax import lax
from jax.experimental import pallas as pl
from jax.experimental.pallas import tpu as pltpu
from jax.experimental.pallas import tpu_sc as plsc

_NC = 2
_NS = 16
_NW = _NC * _NS

_CH = 128
_G = 4


def kernel(input_ids, table):
    B, H = input_ids.shape
    V, D = table.shape
    total = B * H
    per_w = total // _NW
    n_vec = per_w // _CH
    n_grp = n_vec // _G
    grp_rows = _G * _CH

    ids = input_ids.reshape(_NW, n_vec, _CH)

    mesh = plsc.VectorSubcoreMesh(core_axis_name="c", subcore_axis_name="s")

    @functools.partial(
        pl.kernel,
        out_type=jax.ShapeDtypeStruct((total, D), table.dtype),
        mesh=mesh,
        scratch_types=[
            pltpu.VMEM((n_vec, _CH), jnp.int32),
            pltpu.VMEM((grp_rows, D), jnp.float32),
            pltpu.SemaphoreType.DMA,
        ],
        compiler_params=pltpu.CompilerParams(use_tc_tiling_on_sc=False),
    )
    def run(ids_hbm, table_hbm, out_hbm, idx_v, rows_v, sem):
        wid = lax.axis_index("s") * _NC + lax.axis_index("c")
        base = wid * per_w
        pltpu.sync_copy(ids_hbm.at[wid], idx_v)

        def body(g, carry):
            copies = []
            for j in range(_G):
                copies.append(
                    pltpu.async_copy(
                        table_hbm.at[idx_v.at[g * _G + j]],
                        rows_v.at[pl.ds(j * _CH, _CH)],
                        sem,
                    )
                )
            for c in copies:
                c.wait()
            pltpu.sync_copy(rows_v, out_hbm.at[pl.ds(base + g * grp_rows, grp_rows)])
            return carry

        lax.fori_loop(0, n_grp, body, 0)

    out = run(ids, table)
    return out.reshape(B, H, D)


# speedup vs baseline: 1.0260x; 1.0260x over previous
import functools

import jax
import jax.numpy as jnp
from jax import lax
from jax.experimental import pallas as pl
from jax.experimental.pallas import tpu as pltpu
from jax.experimental.pallas import tpu_sc as plsc

_NC = 2
_NS = 16
_NW = _NC * _NS

_CH = 128
_G = 4


def kernel(input_ids, table):
    B, H = input_ids.shape
    V, D = table.shape
    total = B * H
    per_w = total // _NW
    n_vec = per_w // _CH
    n_grp = n_vec // _G
    grp_rows = _G * _CH

    ids = input_ids.reshape(_NW, n_vec, _CH)

    mesh = plsc.VectorSubcoreMesh(core_axis_name="c", subcore_axis_name="s")

    @functools.partial(
        pl.kernel,
        out_type=jax.ShapeDtypeStruct((total, D), table.dtype),
        mesh=mesh,
        scratch_types=[
            pltpu.VMEM((n_vec, _CH), jnp.int32),
            pltpu.VMEM((grp_rows, D), jnp.float32),
            pltpu.VMEM((grp_rows, D), jnp.float32),
            pltpu.SemaphoreType.DMA,
            pltpu.SemaphoreType.DMA,
            pltpu.SemaphoreType.DMA,
            pltpu.SemaphoreType.DMA,
        ],
        compiler_params=pltpu.CompilerParams(use_tc_tiling_on_sc=False),
    )
    def run(ids_hbm, table_hbm, out_hbm, idx_v, rows0, rows1, sg0, sg1, so0, so1):
        wid = lax.axis_index("s") * _NC + lax.axis_index("c")
        base = wid * per_w
        pltpu.sync_copy(ids_hbm.at[wid], idx_v)

        bufs = (rows0, rows1)
        sgs = (sg0, sg1)
        sos = (so0, so1)

        def fire_gather(g, b):
            for j in range(_G):
                pltpu.async_copy(
                    table_hbm.at[idx_v.at[g * _G + j]],
                    bufs[b].at[pl.ds(j * _CH, _CH)],
                    sgs[b],
                )

        def wait_gather(g, b):
            for j in range(_G):
                pltpu.make_async_copy(
                    table_hbm.at[idx_v.at[g * _G + j]],
                    bufs[b].at[pl.ds(j * _CH, _CH)],
                    sgs[b],
                ).wait()

        def fire_out(g, b):
            pltpu.async_copy(
                bufs[b], out_hbm.at[pl.ds(base + g * grp_rows, grp_rows)], sos[b]
            )

        def wait_out(g, b):
            pltpu.make_async_copy(
                bufs[b], out_hbm.at[pl.ds(base + g * grp_rows, grp_rows)], sos[b]
            ).wait()

        fire_gather(0, 0)
        fire_gather(1, 1)
        wait_gather(0, 0)
        fire_out(0, 0)

        def body(i, carry):
            g1 = 2 * i + 1
            wait_out(g1 - 1, 0)
            fire_gather(g1 + 1, 0)
            wait_gather(g1, 1)
            fire_out(g1, 1)
            g2 = 2 * i + 2
            wait_out(g2 - 1, 1)
            fire_gather(g2 + 1, 1)
            wait_gather(g2, 0)
            fire_out(g2, 0)
            return carry

        lax.fori_loop(0, (n_grp - 2) // 2, body, 0)

        g_last = n_grp - 1
        wait_gather(g_last, 1)
        fire_out(g_last, 1)
        wait_out(g_last - 1, 0)
        wait_out(g_last, 1)

    out = run(ids, table)
    return out.reshape(B, H, D)
